# baseline (speedup 1.0000x reference)
import functools

import jax
import jax.numpy as jnp
from jax import lax
from jax.experimental import pallas as pl
from jax.experimental.pallas import tpu as pltpu
from jax.experimental.pallas import tpu_sc as plsc

_B, _H, _W, _D = 16, 32, 32, 256
_NC, _NS, _L = 2, 16, 16


def _sc_map_body(col_hbm, row_hbm, map_hbm, chunk, row_buf, sem):
    i = lax.axis_index("s") * _NC + lax.axis_index("c")
    pltpu.sync_copy(col_hbm.at[pl.ds(0, _W), :], chunk.at[:, pl.ds(0, _D)])
    pltpu.sync_copy(row_hbm.at[i, :], row_buf)
    for t in range(_D // _L):
        v = row_buf[pl.ds(t * _L, _L)]
        for j in range(_W):
            chunk[j, pl.ds(_D + t * _L, _L)] = v
    pltpu.async_copy(chunk, map_hbm.at[i], sem).wait()


def _tc_rep_body(map_ref, out_ref, sems):
    b = out_ref.shape[0]
    copies = [pltpu.make_async_copy(map_ref, out_ref.at[i],
                                    sems.at[i % sems.shape[0]])
              for i in range(b)]
    for c in copies:
        c.start()
    for c in copies:
        c.wait()


def kernel(x, row_embed, col_embed):
    b = x.shape[0]
    h, w = x.shape[-2], x.shape[-1]
    d = row_embed.shape[1]
    mesh = plsc.VectorSubcoreMesh(core_axis_name="c", subcore_axis_name="s")
    build_map = functools.partial(
        pl.kernel,
        mesh=mesh,
        out_type=jax.ShapeDtypeStruct((h, w, 2 * d), jnp.float32),
        scratch_types=[
            pltpu.VMEM((w, 2 * d), jnp.float32),
            pltpu.VMEM((d,), jnp.float32),
            pltpu.SemaphoreType.DMA,
        ],
    )(_sc_map_body)
    pos_map = build_map(col_embed, row_embed)
    out = pl.pallas_call(
        _tc_rep_body,
        in_specs=[pl.BlockSpec(memory_space=pltpu.MemorySpace.VMEM)],
        out_specs=pl.BlockSpec(memory_space=pl.ANY),
        out_shape=jax.ShapeDtypeStruct((b, h, w, 2 * d), jnp.float32),
        scratch_shapes=[pltpu.SemaphoreType.DMA((8,))],
    )(pos_map)
    return jnp.transpose(out, (0, 3, 1, 2))


# speedup vs baseline: 2.5034x; 2.5034x over previous
import jax
import jax.numpy as jnp
from jax.experimental import pallas as pl
from jax.experimental.pallas import tpu as pltpu


def _pos_kernel(col_ref, row_ref, out_ref, scratch, sems):
    h, w, d2 = scratch.shape
    d = d2 // 2
    b = out_ref.shape[0]
    hh = h // 2
    copies = []
    for half in range(2):
        sl = slice(half * hh, (half + 1) * hh)
        scratch[sl, :, :d] = jnp.broadcast_to(
            col_ref[...][None, :, :], (hh, w, d))
        scratch[sl, :, d:] = jnp.broadcast_to(
            row_ref[...][sl][:, None, :], (hh, w, d))
        for i in range(b):
            c = pltpu.make_async_copy(
                scratch.at[sl], out_ref.at[i, sl],
                sems.at[(half * b + i) % sems.shape[0]])
            c.start()
            copies.append(c)
    for c in copies:
        c.wait()


def kernel(x, row_embed, col_embed):
    b = x.shape[0]
    h, w = x.shape[-2], x.shape[-1]
    d = row_embed.shape[1]
    out = pl.pallas_call(
        _pos_kernel,
        in_specs=[
            pl.BlockSpec(memory_space=pltpu.MemorySpace.VMEM),
            pl.BlockSpec(memory_space=pltpu.MemorySpace.VMEM),
        ],
        out_specs=pl.BlockSpec(memory_space=pl.ANY),
        out_shape=jax.ShapeDtypeStruct((b, h, w, 2 * d), jnp.float32),
        scratch_shapes=[
            pltpu.VMEM((h, w, 2 * d), jnp.float32),
            pltpu.SemaphoreType.DMA((8,)),
        ],
    )(col_embed[:w], row_embed[:h])
    return jnp.transpose(out, (0, 3, 1, 2))


# speedup vs baseline: 2.5132x; 1.0039x over previous
import jax
import jax.numpy as jnp
from jax.experimental import pallas as pl
from jax.experimental.pallas import tpu as pltpu


def _pos_kernel(col_ref, row_ref, out_ref, scratch, sems):
    h, w, d2 = scratch.shape
    d = d2 // 2
    b = out_ref.shape[0]
    hh = h // 4
    copies = []
    for half in range(4):
        sl = slice(half * hh, (half + 1) * hh)
        scratch[sl, :, :d] = jnp.broadcast_to(
            col_ref[...][None, :, :], (hh, w, d))
        scratch[sl, :, d:] = jnp.broadcast_to(
            row_ref[...][sl][:, None, :], (hh, w, d))
        for i in range(b):
            c = pltpu.make_async_copy(
                scratch.at[sl], out_ref.at[i, sl],
                sems.at[(half * b + i) % sems.shape[0]])
            c.start()
            copies.append(c)
    for c in copies:
        c.wait()


def kernel(x, row_embed, col_embed):
    b = x.shape[0]
    h, w = x.shape[-2], x.shape[-1]
    d = row_embed.shape[1]
    out = pl.pallas_call(
        _pos_kernel,
        in_specs=[
            pl.BlockSpec(memory_space=pltpu.MemorySpace.VMEM),
            pl.BlockSpec(memory_space=pltpu.MemorySpace.VMEM),
        ],
        out_specs=pl.BlockSpec(memory_space=pl.ANY),
        out_shape=jax.ShapeDtypeStruct((b, h, w, 2 * d), jnp.float32),
        scratch_shapes=[
            pltpu.VMEM((h, w, 2 * d), jnp.float32),
            pltpu.SemaphoreType.DMA((8,)),
        ],
    )(col_embed[:w], row_embed[:h])
    return jnp.transpose(out, (0, 3, 1, 2))
